# Initial kernel scaffold; baseline (speedup 1.0000x reference)
import jax
import jax.numpy as jnp
from jax.experimental import pallas as pl


def kernel(x, edge_index, W1, b1, W2, b2):
    raise NotImplementedError("write your pallas kernel here")


# speedup vs baseline: 141.6651x; 141.6651x over previous
import functools

import jax
import jax.numpy as jnp
from jax import lax
from jax.experimental import pallas as pl
from jax.experimental.pallas import tpu as pltpu
from jax.experimental.pallas import tpu_sc as plsc

L = 16
NC = 2
NS = 16
NW = NC * NS
C = 2048


def _make_sc_pass(np_, epw, gather):
    nch = epw // C
    nps = np_ // NS
    mesh = plsc.VectorSubcoreMesh(core_axis_name="c", subcore_axis_name="s")

    scratch = [
        pltpu.VMEM_SHARED((np_,), jnp.float32),
        pltpu.VMEM((nps,), jnp.float32),
        pltpu.VMEM((C,), jnp.int32),
        pltpu.VMEM((C,), jnp.float32),
    ]
    if gather:
        scratch += [
            pltpu.VMEM((C,), jnp.int32),
            pltpu.VMEM((np_,), jnp.float32),
        ]

    def body(*refs):
        if gather:
            (plane_hbm, src_hbm, dst_hbm, out_hbm,
             acc_sh, wb, dstv, valv, srcv, pv) = refs
        else:
            (dst_hbm, out_hbm, acc_sh, wb, dstv, valv) = refs
        cid = lax.axis_index("c")
        sid = lax.axis_index("s")
        wid = cid * NS + sid

        zeros16 = jnp.zeros((L,), jnp.float32)
        def zb(i, carry):
            wb[pl.ds(i * L, L)] = zeros16
            return carry
        lax.fori_loop(0, nps // L, zb, 0)
        pltpu.sync_copy(wb, acc_sh.at[pl.ds(sid * nps, nps)])

        if gather:
            pltpu.sync_copy(plane_hbm, pv)
        else:
            ones16 = jnp.ones((L,), jnp.float32)
            def ob(i, carry):
                valv[pl.ds(i * L, L)] = ones16
                return carry
            lax.fori_loop(0, C // L, ob, 0)

        plsc.subcore_barrier()

        def chunk(g, carry):
            base = wid * epw + g * C
            pltpu.sync_copy(dst_hbm.at[pl.ds(base, C)], dstv)
            if gather:
                pltpu.sync_copy(src_hbm.at[pl.ds(base, C)], srcv)
                def gb(j, c2):
                    idx = srcv[pl.ds(j * L, L)]
                    valv[pl.ds(j * L, L)] = plsc.load_gather(pv, [idx])
                    return c2
                lax.fori_loop(0, C // L, gb, 0)
            pltpu.sync_copy(valv, acc_sh.at[dstv], add=True)
            return carry
        lax.fori_loop(0, nch, chunk, 0)

        plsc.subcore_barrier()

        pltpu.sync_copy(acc_sh.at[pl.ds(sid * nps, nps)], wb)
        pltpu.sync_copy(wb, out_hbm.at[cid, sid])

    return pl.kernel(
        body,
        out_type=jax.ShapeDtypeStruct((NC, NS, nps), jnp.float32),
        mesh=mesh,
        scratch_types=scratch,
        compiler_params=pltpu.CompilerParams(
            use_tc_tiling_on_sc=False, needs_layout_passes=False
        ),
    )


def _tc_deg_body(dp_ref, xp_ref, dinv_ref, p_ref):
    deg = dp_ref[0] + dp_ref[1] + 1.0
    dinv = lax.rsqrt(deg)
    dinv = dinv * (1.5 - 0.5 * deg * dinv * dinv)
    dinv = dinv * (1.5 - 0.5 * deg * dinv * dinv)
    dinv_ref[...] = dinv
    p_ref[...] = xp_ref[...] * dinv


def _tc_tq_body(k_feats, s1p_ref, dinv_ref, p_ref, w1_ref, b1_ref, w2_ref,
                q0_ref, q1_ref):
    dinv = dinv_ref[...]
    t = dinv * (s1p_ref[0] + s1p_ref[1] + p_ref[...])
    acc0 = jnp.zeros_like(t)
    acc1 = jnp.zeros_like(t)
    for k in range(k_feats):
        h = jnp.maximum(t * w1_ref[0, k] + b1_ref[k], 0.0)
        acc0 = acc0 + h * w2_ref[k, 0]
        acc1 = acc1 + h * w2_ref[k, 1]
    q0_ref[...] = acc0 * dinv
    q1_ref[...] = acc1 * dinv


def _tc_out_body(s20_ref, s21_ref, q0_ref, q1_ref, dinv_ref, b2_ref,
                 o0_ref, o1_ref):
    dinv = dinv_ref[...]
    o0_ref[...] = dinv * (s20_ref[0] + s20_ref[1] + q0_ref[...]) + b2_ref[0]
    o1_ref[...] = dinv * (s21_ref[0] + s21_ref[1] + q1_ref[...]) + b2_ref[1]


def _vspec():
    return pl.BlockSpec(memory_space=pltpu.MemorySpace.VMEM)


def _sspec():
    return pl.BlockSpec(memory_space=pltpu.SMEM)


def kernel(x, edge_index, W1, b1, W2, b2):
    n = x.shape[0]
    e = edge_index.shape[1]
    k_feats = W1.shape[1]

    np_ = (n // 1024 + 1) * 1024
    npr = np_ // 128
    epw = -(-e // (NW * C)) * C
    ep = epw * NW

    src = edge_index[0].astype(jnp.int32)
    dst = edge_index[1].astype(jnp.int32)
    if ep > e:
        pad = (n + jnp.arange(ep - e, dtype=jnp.int32) % (np_ - n))
        src = jnp.concatenate([src, pad])
        dst = jnp.concatenate([dst, pad])

    deg_pass = _make_sc_pass(np_, epw, gather=False)
    seg_pass = _make_sc_pass(np_, epw, gather=True)

    deg_part = deg_pass(dst).reshape(2, npr, 128)

    xp = jnp.pad(x[:, 0], (0, np_ - n)).reshape(npr, 128)
    dinv, p = pl.pallas_call(
        _tc_deg_body,
        out_shape=[jax.ShapeDtypeStruct((npr, 128), jnp.float32)] * 2,
        in_specs=[_vspec(), _vspec()],
        out_specs=[_vspec(), _vspec()],
    )(deg_part, xp)

    s1_part = seg_pass(p.reshape(np_), src, dst).reshape(2, npr, 128)

    q0, q1 = pl.pallas_call(
        functools.partial(_tc_tq_body, k_feats),
        out_shape=[jax.ShapeDtypeStruct((npr, 128), jnp.float32)] * 2,
        in_specs=[_vspec(), _vspec(), _vspec(), _sspec(), _sspec(), _sspec()],
        out_specs=[_vspec(), _vspec()],
    )(s1_part, dinv, p, W1, b1, W2)

    s20_part = seg_pass(q0.reshape(np_), src, dst).reshape(2, npr, 128)
    s21_part = seg_pass(q1.reshape(np_), src, dst).reshape(2, npr, 128)

    o0, o1 = pl.pallas_call(
        _tc_out_body,
        out_shape=[jax.ShapeDtypeStruct((npr, 128), jnp.float32)] * 2,
        in_specs=[_vspec(), _vspec(), _vspec(), _vspec(), _vspec(), _sspec()],
        out_specs=[_vspec(), _vspec()],
    )(s20_part, s21_part, q0, q1, dinv, b2)

    return jnp.stack([o0.reshape(np_)[:n], o1.reshape(np_)[:n]], axis=-1)


# speedup vs baseline: 256.8873x; 1.8133x over previous
import functools

import jax
import jax.numpy as jnp
from jax import lax
from jax.experimental import pallas as pl
from jax.experimental.pallas import tpu as pltpu
from jax.experimental.pallas import tpu_sc as plsc

L = 16
NC = 2
NS = 16
NW = NC * NS
C = 2048
NBUF = 2


def _fill(ref, n, vec):
    def b(i, carry):
        ref[pl.ds(i * L, L)] = vec
        return carry
    lax.fori_loop(0, n // L, b, 0)


def _make_deg_pass(np_, epw):
    nch = epw // C
    mesh = plsc.VectorSubcoreMesh(core_axis_name="c", subcore_axis_name="s")

    def body(dst_hbm, out_hbm, accv, dstv, sem):
        cid = lax.axis_index("c")
        sid = lax.axis_index("s")
        wid = cid * NS + sid

        _fill(accv, np_, jnp.zeros((L,), jnp.float32))
        ones16 = jnp.ones((L,), jnp.float32)

        def start_in(g, slot):
            base = wid * epw + g * C
            pltpu.async_copy(dst_hbm.at[pl.ds(base, C)], dstv.at[slot],
                             sem.at[slot])

        def wait_in(slot):
            pltpu.make_async_copy(dst_hbm.at[pl.ds(0, C)], dstv.at[slot],
                                  sem.at[slot]).wait()

        start_in(0, 0)
        start_in(1, 1)

        def outer(h, carry):
            for b in range(NBUF):
                g = h * NBUF + b
                wait_in(b)
                def gb(j, c2):
                    idx = dstv.at[b][pl.ds(j * L, L)]
                    plsc.addupdate_scatter(accv, [idx], ones16)
                    return c2
                lax.fori_loop(0, C // L, gb, 0)
                nxt = g + NBUF

                @pl.when(nxt < nch)
                def _():
                    start_in(nxt, b)
            return carry
        lax.fori_loop(0, nch // NBUF, outer, 0)

        pltpu.sync_copy(accv, out_hbm.at[wid])

    return pl.kernel(
        body,
        out_type=jax.ShapeDtypeStruct((NW, np_), jnp.float32),
        mesh=mesh,
        scratch_types=[
            pltpu.VMEM((np_,), jnp.float32),
            pltpu.VMEM((NBUF, C), jnp.int32),
            pltpu.SemaphoreType.DMA((NBUF,)),
        ],
        compiler_params=pltpu.CompilerParams(
            use_tc_tiling_on_sc=False, needs_layout_passes=False
        ),
    )


def _make_seg_pass(np_, epw):
    nch = epw // C
    nps = np_ // NS
    mesh = plsc.VectorSubcoreMesh(core_axis_name="c", subcore_axis_name="s")

    def body(plane_hbm, src_hbm, dst_hbm, out_hbm,
             acc_sh, wb, srcv, dstv, dsto, valv, pv, sem_s, sem_d, sem_o):
        cid = lax.axis_index("c")
        sid = lax.axis_index("s")
        wid = cid * NS + sid

        _fill(wb, nps, jnp.zeros((L,), jnp.float32))
        pltpu.sync_copy(wb, acc_sh.at[pl.ds(sid * nps, nps)])
        pltpu.sync_copy(plane_hbm, pv)
        plsc.subcore_barrier()

        def start_in(g, slot):
            base = wid * epw + g * C
            pltpu.async_copy(src_hbm.at[pl.ds(base, C)], srcv.at[slot],
                             sem_s.at[slot])
            pltpu.async_copy(dst_hbm.at[pl.ds(base, C)], dstv.at[slot],
                             sem_d.at[slot])

        def wait_in(slot):
            pltpu.make_async_copy(src_hbm.at[pl.ds(0, C)], srcv.at[slot],
                                  sem_s.at[slot]).wait()
            pltpu.make_async_copy(dst_hbm.at[pl.ds(0, C)], dstv.at[slot],
                                  sem_d.at[slot]).wait()

        def wait_out(slot):
            pltpu.make_async_copy(valv.at[slot], acc_sh.at[dsto.at[slot]],
                                  sem_o.at[slot]).wait()

        start_in(0, 0)
        start_in(1, 1)

        def outer(h, carry):
            for b in range(NBUF):
                g = h * NBUF + b
                wait_in(b)

                @pl.when(g >= NBUF)
                def _():
                    wait_out(b)

                def gb(j, c2):
                    sl = pl.ds(j * L, L)
                    idx = srcv.at[b][sl]
                    valv.at[b][sl] = plsc.load_gather(pv, [idx])
                    dsto.at[b][sl] = dstv.at[b][sl]
                    return c2
                lax.fori_loop(0, C // L, gb, 0)

                pltpu.async_copy(valv.at[b], acc_sh.at[dsto.at[b]],
                                 sem_o.at[b], add=True)
                nxt = g + NBUF

                @pl.when(nxt < nch)
                def _():
                    start_in(nxt, b)
            return carry
        lax.fori_loop(0, nch // NBUF, outer, 0)

        wait_out(0)
        wait_out(1)
        plsc.subcore_barrier()

        pltpu.sync_copy(acc_sh.at[pl.ds(sid * nps, nps)], wb)
        pltpu.sync_copy(wb, out_hbm.at[cid, sid])

    return pl.kernel(
        body,
        out_type=jax.ShapeDtypeStruct((NC, NS, nps), jnp.float32),
        mesh=mesh,
        scratch_types=[
            pltpu.VMEM_SHARED((np_,), jnp.float32),
            pltpu.VMEM((nps,), jnp.float32),
            pltpu.VMEM((NBUF, C), jnp.int32),
            pltpu.VMEM((NBUF, C), jnp.int32),
            pltpu.VMEM((NBUF, C), jnp.int32),
            pltpu.VMEM((NBUF, C), jnp.float32),
            pltpu.VMEM((np_,), jnp.float32),
            pltpu.SemaphoreType.DMA((NBUF,)),
            pltpu.SemaphoreType.DMA((NBUF,)),
            pltpu.SemaphoreType.DMA((NBUF,)),
        ],
        compiler_params=pltpu.CompilerParams(
            use_tc_tiling_on_sc=False, needs_layout_passes=False
        ),
    )


def _tc_deg_body(nw, dp_ref, xp_ref, dinv_ref, p_ref):
    deg = dp_ref[0]
    for w in range(1, nw):
        deg = deg + dp_ref[w]
    deg = deg + 1.0
    dinv = lax.rsqrt(deg)
    dinv = dinv * (1.5 - 0.5 * deg * dinv * dinv)
    dinv = dinv * (1.5 - 0.5 * deg * dinv * dinv)
    dinv_ref[...] = dinv
    p_ref[...] = xp_ref[...] * dinv


def _tc_tq_body(k_feats, s1p_ref, dinv_ref, p_ref, w1_ref, b1_ref, w2_ref,
                q0_ref, q1_ref):
    dinv = dinv_ref[...]
    t = dinv * (s1p_ref[0] + s1p_ref[1] + p_ref[...])
    acc0 = jnp.zeros_like(t)
    acc1 = jnp.zeros_like(t)
    for k in range(k_feats):
        h = jnp.maximum(t * w1_ref[0, k] + b1_ref[k], 0.0)
        acc0 = acc0 + h * w2_ref[k, 0]
        acc1 = acc1 + h * w2_ref[k, 1]
    q0_ref[...] = acc0 * dinv
    q1_ref[...] = acc1 * dinv


def _tc_out_body(s20_ref, s21_ref, q0_ref, q1_ref, dinv_ref, b2_ref,
                 o0_ref, o1_ref):
    dinv = dinv_ref[...]
    o0_ref[...] = dinv * (s20_ref[0] + s20_ref[1] + q0_ref[...]) + b2_ref[0]
    o1_ref[...] = dinv * (s21_ref[0] + s21_ref[1] + q1_ref[...]) + b2_ref[1]


def _vspec():
    return pl.BlockSpec(memory_space=pltpu.MemorySpace.VMEM)


def _sspec():
    return pl.BlockSpec(memory_space=pltpu.SMEM)


def kernel(x, edge_index, W1, b1, W2, b2):
    n = x.shape[0]
    e = edge_index.shape[1]
    k_feats = W1.shape[1]

    np_ = (n // 1024 + 1) * 1024
    npr = np_ // 128
    epw = -(-e // (NW * C * NBUF)) * C * NBUF
    ep = epw * NW

    src = edge_index[0].astype(jnp.int32)
    dst = edge_index[1].astype(jnp.int32)
    if ep > e:
        pad = (n + jnp.arange(ep - e, dtype=jnp.int32) % (np_ - n))
        src = jnp.concatenate([src, pad])
        dst = jnp.concatenate([dst, pad])

    deg_pass = _make_deg_pass(np_, epw)
    seg_pass = _make_seg_pass(np_, epw)

    deg_part = deg_pass(dst).reshape(NW, npr, 128)

    xp = jnp.pad(x[:, 0], (0, np_ - n)).reshape(npr, 128)
    dinv, p = pl.pallas_call(
        functools.partial(_tc_deg_body, NW),
        out_shape=[jax.ShapeDtypeStruct((npr, 128), jnp.float32)] * 2,
        in_specs=[_vspec(), _vspec()],
        out_specs=[_vspec(), _vspec()],
    )(deg_part, xp)

    s1_part = seg_pass(p.reshape(np_), src, dst).reshape(2, npr, 128)

    q0, q1 = pl.pallas_call(
        functools.partial(_tc_tq_body, k_feats),
        out_shape=[jax.ShapeDtypeStruct((npr, 128), jnp.float32)] * 2,
        in_specs=[_vspec(), _vspec(), _vspec(), _sspec(), _sspec(), _sspec()],
        out_specs=[_vspec(), _vspec()],
    )(s1_part, dinv, p, W1, b1, W2)

    s20_part = seg_pass(q0.reshape(np_), src, dst).reshape(2, npr, 128)
    s21_part = seg_pass(q1.reshape(np_), src, dst).reshape(2, npr, 128)

    o0, o1 = pl.pallas_call(
        _tc_out_body,
        out_shape=[jax.ShapeDtypeStruct((npr, 128), jnp.float32)] * 2,
        in_specs=[_vspec(), _vspec(), _vspec(), _vspec(), _vspec(), _sspec()],
        out_specs=[_vspec(), _vspec()],
    )(s20_part, s21_part, q0, q1, dinv, b2)

    return jnp.stack([o0.reshape(np_)[:n], o1.reshape(np_)[:n]], axis=-1)


# speedup vs baseline: 290.0557x; 1.1291x over previous
import functools

import jax
import jax.numpy as jnp
from jax import lax
from jax.experimental import pallas as pl
from jax.experimental.pallas import tpu as pltpu
from jax.experimental.pallas import tpu_sc as plsc

L = 16
NC = 2
NS = 16
NW = NC * NS
C = 2000
NBUF = 2


def _fill(ref, n, vec):
    def b(i, carry):
        ref[pl.ds(i * L, L)] = vec
        return carry
    lax.fori_loop(0, n // L, b, 0)


def _make_deg_pass(np_, epw):
    nch = epw // C
    mesh = plsc.VectorSubcoreMesh(core_axis_name="c", subcore_axis_name="s")

    def body(dst_hbm, out_hbm, accv, dstv, sem):
        cid = lax.axis_index("c")
        sid = lax.axis_index("s")
        wid = cid * NS + sid

        _fill(accv, np_, jnp.zeros((L,), jnp.float32))
        ones16 = jnp.ones((L,), jnp.float32)

        def start_in(g, slot):
            base = wid * epw + g * C
            pltpu.async_copy(dst_hbm.at[pl.ds(base, C)], dstv.at[slot],
                             sem.at[slot])

        def wait_in(slot):
            pltpu.make_async_copy(dst_hbm.at[pl.ds(0, C)], dstv.at[slot],
                                  sem.at[slot]).wait()

        start_in(0, 0)
        start_in(1, 1)

        def outer(h, carry):
            for b in range(NBUF):
                g = h * NBUF + b
                wait_in(b)
                for j in range(C // L):
                    idx = dstv.at[b][pl.ds(j * L, L)]
                    plsc.addupdate_scatter(accv, [idx], ones16)
                nxt = g + NBUF

                @pl.when(nxt < nch)
                def _():
                    start_in(nxt, b)
            return carry
        lax.fori_loop(0, nch // NBUF, outer, 0)

        pltpu.sync_copy(accv, out_hbm.at[wid])

    return pl.kernel(
        body,
        out_type=jax.ShapeDtypeStruct((NW, np_), jnp.float32),
        mesh=mesh,
        scratch_types=[
            pltpu.VMEM((np_,), jnp.float32),
            pltpu.VMEM((NBUF, C), jnp.int32),
            pltpu.SemaphoreType.DMA((NBUF,)),
        ],
        compiler_params=pltpu.CompilerParams(
            use_tc_tiling_on_sc=False, needs_layout_passes=False
        ),
    )


def _make_seg_pass(np_, epw):
    nch = epw // C
    nps = np_ // NS
    mesh = plsc.VectorSubcoreMesh(core_axis_name="c", subcore_axis_name="s")

    def body(plane_hbm, src_hbm, dst_hbm, out_hbm,
             acc_sh, wb, srcv, dstv, dsto, valv, pv, sem_s, sem_d, sem_o):
        cid = lax.axis_index("c")
        sid = lax.axis_index("s")
        wid = cid * NS + sid

        _fill(wb, nps, jnp.zeros((L,), jnp.float32))
        pltpu.sync_copy(wb, acc_sh.at[pl.ds(sid * nps, nps)])
        pltpu.sync_copy(plane_hbm, pv)
        plsc.subcore_barrier()

        def start_in(g, slot):
            base = wid * epw + g * C
            pltpu.async_copy(src_hbm.at[pl.ds(base, C)], srcv.at[slot],
                             sem_s.at[slot])
            pltpu.async_copy(dst_hbm.at[pl.ds(base, C)], dstv.at[slot],
                             sem_d.at[slot])

        def wait_in(slot):
            pltpu.make_async_copy(src_hbm.at[pl.ds(0, C)], srcv.at[slot],
                                  sem_s.at[slot]).wait()
            pltpu.make_async_copy(dst_hbm.at[pl.ds(0, C)], dstv.at[slot],
                                  sem_d.at[slot]).wait()

        def wait_out(slot):
            pltpu.make_async_copy(valv.at[slot], acc_sh.at[dsto.at[slot]],
                                  sem_o.at[slot]).wait()

        start_in(0, 0)
        start_in(1, 1)

        def outer(h, carry):
            for b in range(NBUF):
                g = h * NBUF + b
                wait_in(b)

                @pl.when(g >= NBUF)
                def _():
                    wait_out(b)

                for j in range(C // L):
                    sl = pl.ds(j * L, L)
                    idx = srcv.at[b][sl]
                    valv.at[b][sl] = plsc.load_gather(pv, [idx])
                    dsto.at[b][sl] = dstv.at[b][sl]

                pltpu.async_copy(valv.at[b], acc_sh.at[dsto.at[b]],
                                 sem_o.at[b], add=True)
                nxt = g + NBUF

                @pl.when(nxt < nch)
                def _():
                    start_in(nxt, b)
            return carry
        lax.fori_loop(0, nch // NBUF, outer, 0)

        wait_out(0)
        wait_out(1)
        plsc.subcore_barrier()

        pltpu.sync_copy(acc_sh.at[pl.ds(sid * nps, nps)], wb)
        pltpu.sync_copy(wb, out_hbm.at[cid, sid])

    return pl.kernel(
        body,
        out_type=jax.ShapeDtypeStruct((NC, NS, nps), jnp.float32),
        mesh=mesh,
        scratch_types=[
            pltpu.VMEM_SHARED((np_,), jnp.float32),
            pltpu.VMEM((nps,), jnp.float32),
            pltpu.VMEM((NBUF, C), jnp.int32),
            pltpu.VMEM((NBUF, C), jnp.int32),
            pltpu.VMEM((NBUF, C), jnp.int32),
            pltpu.VMEM((NBUF, C), jnp.float32),
            pltpu.VMEM((np_,), jnp.float32),
            pltpu.SemaphoreType.DMA((NBUF,)),
            pltpu.SemaphoreType.DMA((NBUF,)),
            pltpu.SemaphoreType.DMA((NBUF,)),
        ],
        compiler_params=pltpu.CompilerParams(
            use_tc_tiling_on_sc=False, needs_layout_passes=False
        ),
    )


def _tc_deg_body(nw, dp_ref, xp_ref, dinv_ref, p_ref):
    deg = dp_ref[0]
    for w in range(1, nw):
        deg = deg + dp_ref[w]
    deg = deg + 1.0
    dinv = lax.rsqrt(deg)
    dinv = dinv * (1.5 - 0.5 * deg * dinv * dinv)
    dinv = dinv * (1.5 - 0.5 * deg * dinv * dinv)
    dinv_ref[...] = dinv
    p_ref[...] = xp_ref[...] * dinv


def _tc_tq_body(k_feats, s1p_ref, dinv_ref, p_ref, w1_ref, b1_ref, w2_ref,
                q0_ref, q1_ref):
    dinv = dinv_ref[...]
    t = dinv * (s1p_ref[0] + s1p_ref[1] + p_ref[...])
    acc0 = jnp.zeros_like(t)
    acc1 = jnp.zeros_like(t)
    for k in range(k_feats):
        h = jnp.maximum(t * w1_ref[0, k] + b1_ref[k], 0.0)
        acc0 = acc0 + h * w2_ref[k, 0]
        acc1 = acc1 + h * w2_ref[k, 1]
    q0_ref[...] = acc0 * dinv
    q1_ref[...] = acc1 * dinv


def _tc_out_body(s20_ref, s21_ref, q0_ref, q1_ref, dinv_ref, b2_ref,
                 o0_ref, o1_ref):
    dinv = dinv_ref[...]
    o0_ref[...] = dinv * (s20_ref[0] + s20_ref[1] + q0_ref[...]) + b2_ref[0]
    o1_ref[...] = dinv * (s21_ref[0] + s21_ref[1] + q1_ref[...]) + b2_ref[1]


def _vspec():
    return pl.BlockSpec(memory_space=pltpu.MemorySpace.VMEM)


def _sspec():
    return pl.BlockSpec(memory_space=pltpu.SMEM)


def kernel(x, edge_index, W1, b1, W2, b2):
    n = x.shape[0]
    e = edge_index.shape[1]
    k_feats = W1.shape[1]

    np_ = (n // 1024 + 1) * 1024
    npr = np_ // 128
    epw = -(-e // (NW * C * NBUF)) * C * NBUF
    ep = epw * NW

    src = edge_index[0].astype(jnp.int32)
    dst = edge_index[1].astype(jnp.int32)
    if ep > e:
        pad = (n + jnp.arange(ep - e, dtype=jnp.int32) % (np_ - n))
        src = jnp.concatenate([src, pad])
        dst = jnp.concatenate([dst, pad])

    deg_pass = _make_deg_pass(np_, epw)
    seg_pass = _make_seg_pass(np_, epw)

    deg_part = deg_pass(dst).reshape(NW, npr, 128)

    xp = jnp.pad(x[:, 0], (0, np_ - n)).reshape(npr, 128)
    dinv, p = pl.pallas_call(
        functools.partial(_tc_deg_body, NW),
        out_shape=[jax.ShapeDtypeStruct((npr, 128), jnp.float32)] * 2,
        in_specs=[_vspec(), _vspec()],
        out_specs=[_vspec(), _vspec()],
    )(deg_part, xp)

    s1_part = seg_pass(p.reshape(np_), src, dst).reshape(2, npr, 128)

    q0, q1 = pl.pallas_call(
        functools.partial(_tc_tq_body, k_feats),
        out_shape=[jax.ShapeDtypeStruct((npr, 128), jnp.float32)] * 2,
        in_specs=[_vspec(), _vspec(), _vspec(), _sspec(), _sspec(), _sspec()],
        out_specs=[_vspec(), _vspec()],
    )(s1_part, dinv, p, W1, b1, W2)

    s20_part = seg_pass(q0.reshape(np_), src, dst).reshape(2, npr, 128)
    s21_part = seg_pass(q1.reshape(np_), src, dst).reshape(2, npr, 128)

    o0, o1 = pl.pallas_call(
        _tc_out_body,
        out_shape=[jax.ShapeDtypeStruct((npr, 128), jnp.float32)] * 2,
        in_specs=[_vspec(), _vspec(), _vspec(), _vspec(), _vspec(), _sspec()],
        out_specs=[_vspec(), _vspec()],
    )(s20_part, s21_part, q0, q1, dinv, b2)

    return jnp.stack([o0.reshape(np_)[:n], o1.reshape(np_)[:n]], axis=-1)


# speedup vs baseline: 321.6274x; 1.1088x over previous
import functools

import jax
import jax.numpy as jnp
from jax import lax
from jax.experimental import pallas as pl
from jax.experimental.pallas import tpu as pltpu
from jax.experimental.pallas import tpu_sc as plsc

L = 16
NC = 2
NS = 16
NW = NC * NS
C = 2000
NBUF = 2


def _fill(ref, n, vec):
    if n % (8 * L) == 0:
        def b8(i, carry):
            base = i * (8 * L)
            for k in range(8):
                ref[pl.ds(base + k * L, L)] = vec
            return carry
        lax.fori_loop(0, n // (8 * L), b8, 0)
    else:
        def b(i, carry):
            ref[pl.ds(i * L, L)] = vec
            return carry
        lax.fori_loop(0, n // L, b, 0)


def _make_deg_pass(np_, epw):
    nch = epw // C
    mesh = plsc.VectorSubcoreMesh(core_axis_name="c", subcore_axis_name="s")

    def body(dst_hbm, out_hbm, accv, dstv, sem):
        cid = lax.axis_index("c")
        sid = lax.axis_index("s")
        wid = cid * NS + sid

        _fill(accv, np_, jnp.zeros((L,), jnp.float32))
        ones16 = jnp.ones((L,), jnp.float32)

        def start_in(g, slot):
            base = wid * epw + g * C
            pltpu.async_copy(dst_hbm.at[pl.ds(base, C)], dstv.at[slot],
                             sem.at[slot])

        def wait_in(slot):
            pltpu.make_async_copy(dst_hbm.at[pl.ds(0, C)], dstv.at[slot],
                                  sem.at[slot]).wait()

        start_in(0, 0)
        start_in(1, 1)

        def outer(h, carry):
            for b in range(NBUF):
                g = h * NBUF + b
                wait_in(b)
                for j in range(C // L):
                    idx = dstv.at[b][pl.ds(j * L, L)]
                    plsc.addupdate_scatter(accv, [idx], ones16)
                nxt = g + NBUF

                @pl.when(nxt < nch)
                def _():
                    start_in(nxt, b)
            return carry
        lax.fori_loop(0, nch // NBUF, outer, 0)

        pltpu.sync_copy(accv, out_hbm.at[wid])

    return pl.kernel(
        body,
        out_type=jax.ShapeDtypeStruct((NW, np_), jnp.float32),
        mesh=mesh,
        scratch_types=[
            pltpu.VMEM((np_,), jnp.float32),
            pltpu.VMEM((NBUF, C), jnp.int32),
            pltpu.SemaphoreType.DMA((NBUF,)),
        ],
        compiler_params=pltpu.CompilerParams(
            use_tc_tiling_on_sc=False, needs_layout_passes=False
        ),
    )


def _make_seg_pass(np_, epw):
    nch = epw // C
    nps = np_ // NS
    mesh = plsc.VectorSubcoreMesh(core_axis_name="c", subcore_axis_name="s")

    def body(plane_hbm, src_hbm, dst_hbm, out_hbm,
             acc_sh, wb, srcv, dstv, dsto, valv, pv, sem_s, sem_d, sem_o):
        cid = lax.axis_index("c")
        sid = lax.axis_index("s")
        wid = cid * NS + sid

        _fill(wb, nps, jnp.zeros((L,), jnp.float32))
        pltpu.sync_copy(wb, acc_sh.at[pl.ds(sid * nps, nps)])
        pltpu.sync_copy(plane_hbm, pv)
        plsc.subcore_barrier()

        def start_in(g, slot):
            base = wid * epw + g * C
            pltpu.async_copy(src_hbm.at[pl.ds(base, C)], srcv.at[slot],
                             sem_s.at[slot])
            pltpu.async_copy(dst_hbm.at[pl.ds(base, C)], dstv.at[slot],
                             sem_d.at[slot])

        def wait_in(slot):
            pltpu.make_async_copy(src_hbm.at[pl.ds(0, C)], srcv.at[slot],
                                  sem_s.at[slot]).wait()
            pltpu.make_async_copy(dst_hbm.at[pl.ds(0, C)], dstv.at[slot],
                                  sem_d.at[slot]).wait()

        def wait_out(slot):
            pltpu.make_async_copy(valv.at[slot], acc_sh.at[dsto.at[slot]],
                                  sem_o.at[slot]).wait()

        start_in(0, 0)
        start_in(1, 1)

        def outer(h, carry):
            for b in range(NBUF):
                g = h * NBUF + b
                wait_in(b)

                @pl.when(g >= NBUF)
                def _():
                    wait_out(b)

                for j in range(C // L):
                    sl = pl.ds(j * L, L)
                    idx = srcv.at[b][sl]
                    valv.at[b][sl] = plsc.load_gather(pv, [idx])
                    dsto.at[b][sl] = dstv.at[b][sl]

                pltpu.async_copy(valv.at[b], acc_sh.at[dsto.at[b]],
                                 sem_o.at[b], add=True)
                nxt = g + NBUF

                @pl.when(nxt < nch)
                def _():
                    start_in(nxt, b)
            return carry
        lax.fori_loop(0, nch // NBUF, outer, 0)

        wait_out(0)
        wait_out(1)
        plsc.subcore_barrier()

        pltpu.sync_copy(acc_sh.at[pl.ds(sid * nps, nps)], wb)
        pltpu.sync_copy(wb, out_hbm.at[cid, sid])

    return pl.kernel(
        body,
        out_type=jax.ShapeDtypeStruct((NC, NS, nps), jnp.float32),
        mesh=mesh,
        scratch_types=[
            pltpu.VMEM_SHARED((np_,), jnp.float32),
            pltpu.VMEM((nps,), jnp.float32),
            pltpu.VMEM((NBUF, C), jnp.int32),
            pltpu.VMEM((NBUF, C), jnp.int32),
            pltpu.VMEM((NBUF, C), jnp.int32),
            pltpu.VMEM((NBUF, C), jnp.float32),
            pltpu.VMEM((np_,), jnp.float32),
            pltpu.SemaphoreType.DMA((NBUF,)),
            pltpu.SemaphoreType.DMA((NBUF,)),
            pltpu.SemaphoreType.DMA((NBUF,)),
        ],
        compiler_params=pltpu.CompilerParams(
            use_tc_tiling_on_sc=False, needs_layout_passes=False
        ),
    )


def _tc_deg_body(nw, dp_ref, xp_ref, dinv_ref, p_ref):
    deg = dp_ref[0]
    for w in range(1, nw):
        deg = deg + dp_ref[w]
    deg = deg + 1.0
    dinv = lax.rsqrt(deg)
    dinv = dinv * (1.5 - 0.5 * deg * dinv * dinv)
    dinv = dinv * (1.5 - 0.5 * deg * dinv * dinv)
    dinv_ref[...] = dinv
    p_ref[...] = xp_ref[...] * dinv


def _tc_tq_body(k_feats, s1p_ref, dinv_ref, p_ref, w1_ref, b1_ref, w2_ref,
                q0_ref, q1_ref):
    dinv = dinv_ref[...]
    t = dinv * (s1p_ref[0] + s1p_ref[1] + p_ref[...])
    acc0 = jnp.zeros_like(t)
    acc1 = jnp.zeros_like(t)
    for k in range(k_feats):
        h = jnp.maximum(t * w1_ref[0, k] + b1_ref[k], 0.0)
        acc0 = acc0 + h * w2_ref[k, 0]
        acc1 = acc1 + h * w2_ref[k, 1]
    q0_ref[...] = acc0 * dinv
    q1_ref[...] = acc1 * dinv


def _tc_out_body(s20_ref, s21_ref, q0_ref, q1_ref, dinv_ref, b2_ref,
                 o0_ref, o1_ref):
    dinv = dinv_ref[...]
    o0_ref[...] = dinv * (s20_ref[0] + s20_ref[1] + q0_ref[...]) + b2_ref[0]
    o1_ref[...] = dinv * (s21_ref[0] + s21_ref[1] + q1_ref[...]) + b2_ref[1]


def _vspec():
    return pl.BlockSpec(memory_space=pltpu.MemorySpace.VMEM)


def _sspec():
    return pl.BlockSpec(memory_space=pltpu.SMEM)


def kernel(x, edge_index, W1, b1, W2, b2):
    n = x.shape[0]
    e = edge_index.shape[1]
    k_feats = W1.shape[1]

    np_ = (n // 1024 + 1) * 1024
    npr = np_ // 128
    epw = -(-e // (NW * C * NBUF)) * C * NBUF
    ep = epw * NW

    src = edge_index[0].astype(jnp.int32)
    dst = edge_index[1].astype(jnp.int32)
    if ep > e:
        pad = (n + jnp.arange(ep - e, dtype=jnp.int32) % (np_ - n))
        src = jnp.concatenate([src, pad])
        dst = jnp.concatenate([dst, pad])

    deg_pass = _make_deg_pass(np_, epw)
    seg_pass = _make_seg_pass(np_, epw)

    deg_part = deg_pass(dst).reshape(NW, npr, 128)

    xp = jnp.pad(x[:, 0], (0, np_ - n)).reshape(npr, 128)
    dinv, p = pl.pallas_call(
        functools.partial(_tc_deg_body, NW),
        out_shape=[jax.ShapeDtypeStruct((npr, 128), jnp.float32)] * 2,
        in_specs=[_vspec(), _vspec()],
        out_specs=[_vspec(), _vspec()],
    )(deg_part, xp)

    s1_part = seg_pass(p.reshape(np_), src, dst).reshape(2, npr, 128)

    q0, q1 = pl.pallas_call(
        functools.partial(_tc_tq_body, k_feats),
        out_shape=[jax.ShapeDtypeStruct((npr, 128), jnp.float32)] * 2,
        in_specs=[_vspec(), _vspec(), _vspec(), _sspec(), _sspec(), _sspec()],
        out_specs=[_vspec(), _vspec()],
    )(s1_part, dinv, p, W1, b1, W2)

    s20_part = seg_pass(q0.reshape(np_), src, dst).reshape(2, npr, 128)
    s21_part = seg_pass(q1.reshape(np_), src, dst).reshape(2, npr, 128)

    o0, o1 = pl.pallas_call(
        _tc_out_body,
        out_shape=[jax.ShapeDtypeStruct((npr, 128), jnp.float32)] * 2,
        in_specs=[_vspec(), _vspec(), _vspec(), _vspec(), _vspec(), _sspec()],
        out_specs=[_vspec(), _vspec()],
    )(s20_part, s21_part, q0, q1, dinv, b2)

    return jnp.stack([o0.reshape(np_)[:n], o1.reshape(np_)[:n]], axis=-1)
